# Initial kernel scaffold; baseline (speedup 1.0000x reference)
import jax
import jax.numpy as jnp
from jax.experimental import pallas as pl


def kernel(x, adj, batch, W1, b1, W2, b2, Wm, bm):
    raise NotImplementedError("write your pallas kernel here")


# speedup vs baseline: 20.1863x; 20.1863x over previous
import dataclasses
import functools

import jax
import jax.numpy as jnp
from jax import lax
from jax.experimental import pallas as pl
from jax.experimental.pallas import tpu as pltpu
from jax.experimental.pallas import tpu_sc as plsc

NN = 10000
EE = 320000
FF = 128
NG = 64
NC, NS = 2, 16
WORKERS = NC * NS
CH = 80
NCHUNK = EE // CH
CPW = NCHUNK // WORKERS
EPW = EE // WORKERS
NP = 10240
RPS = NP // NS
RB = CH
TCB = 1000
NB = NN // TCB


def _vmesh():
    return plsc.VectorSubcoreMesh(core_axis_name="c", subcore_axis_name="s")


def _sc_params():
    cp = pltpu.CompilerParams()
    if "needs_layout_passes" in pltpu.CompilerParams.__dataclass_fields__:
        cp = dataclasses.replace(cp, needs_layout_passes=False)
    return cp


def _sc_deg(dst_flat):
    @functools.partial(
        pl.kernel,
        out_type=jax.ShapeDtypeStruct((WORKERS, NN), jnp.float32),
        mesh=_vmesh(),
        compiler_params=_sc_params(),
        scratch_types=[
            pltpu.VMEM((EPW,), jnp.int32),
            pltpu.VMEM((NN,), jnp.float32),
        ],
    )
    def k(dst_hbm, out_hbm, idx_v, hist_v):
        c = lax.axis_index("c")
        s = lax.axis_index("s")
        wid = s * NC + c

        @pl.loop(0, NN // 16)
        def _(i):
            hist_v[pl.ds(i * 16, 16)] = jnp.zeros((16,), jnp.float32)

        pltpu.sync_copy(dst_hbm.at[pl.ds(wid * EPW, EPW)], idx_v)
        ones = jnp.ones((16,), jnp.float32)

        @pl.loop(0, EPW // 16)
        def _(i):
            idx = idx_v[pl.ds(i * 16, 16)]
            plsc.addupdate_scatter(hist_v, [idx], ones)

        pltpu.sync_copy(hist_v, out_hbm.at[wid])

    return k(dst_flat)


def _sc_agg(y, src3d, dst3d):
    @functools.partial(
        pl.kernel,
        out_type=jax.ShapeDtypeStruct((NC, NP, FF), jnp.float32),
        mesh=_vmesh(),
        scratch_types=[
            pltpu.VMEM((CPW, CH), jnp.int32),
            pltpu.VMEM((CPW, CH), jnp.int32),
            pltpu.VMEM((CH, FF), jnp.float32),
            pltpu.VMEM_SHARED((NP, FF), jnp.float32),
        ],
    )
    def k(y_hbm, src_hbm, dst_hbm, out_hbm, src_v, dst_v, gbuf, acc):
        c = lax.axis_index("c")
        s = lax.axis_index("s")
        wid = s * NC + c

        @pl.loop(0, RB)
        def _(r):
            @pl.loop(0, FF // 16)
            def _(q):
                gbuf[r, pl.ds(q * 16, 16)] = jnp.zeros((16,), jnp.float32)

        @pl.loop(0, RPS // RB)
        def _(t):
            pltpu.sync_copy(gbuf, acc.at[pl.ds(s * RPS + t * RB, RB)])

        plsc.subcore_barrier()

        pltpu.sync_copy(src_hbm.at[wid], src_v)
        pltpu.sync_copy(dst_hbm.at[wid], dst_v)

        @pl.loop(0, CPW)
        def _(j):
            pltpu.sync_copy(y_hbm.at[src_v.at[j]], gbuf)
            pltpu.sync_copy(gbuf, acc.at[dst_v.at[j]], add=True)

        plsc.subcore_barrier()

        @pl.loop(0, RPS // RB)
        def _(t):
            pltpu.sync_copy(
                acc.at[pl.ds(s * RPS + t * RB, RB)],
                out_hbm.at[c, pl.ds(s * RPS + t * RB, RB)],
            )

    return k(y, src3d, dst3d)


def _tc_matmul(x, w):
    def body(x_ref, w_ref, o_ref):
        o_ref[...] = jnp.dot(x_ref[...], w_ref[...],
                             preferred_element_type=jnp.float32)

    return pl.pallas_call(
        body,
        grid=(NB,),
        in_specs=[
            pl.BlockSpec((TCB, FF), lambda i: (i, 0)),
            pl.BlockSpec((FF, FF), lambda i: (0, 0)),
        ],
        out_specs=pl.BlockSpec((TCB, FF), lambda i: (i, 0)),
        out_shape=jax.ShapeDtypeStruct((NN, FF), jnp.float32),
    )(x, w)


def _tc_prep(hist, xw1):
    def body(h_ref, xw_ref, dinv_ref, y_ref):
        deg = jnp.sum(h_ref[0], axis=0) + 1.0
        dinv = lax.rsqrt(deg)
        dinv_ref[...] = dinv[:, None]
        y_ref[...] = dinv[:, None] * xw_ref[...]

    return pl.pallas_call(
        body,
        grid=(NB,),
        in_specs=[
            pl.BlockSpec((1, WORKERS, TCB), lambda i: (i, 0, 0)),
            pl.BlockSpec((TCB, FF), lambda i: (i, 0)),
        ],
        out_specs=[
            pl.BlockSpec((TCB, 1), lambda i: (i, 0)),
            pl.BlockSpec((TCB, FF), lambda i: (i, 0)),
        ],
        out_shape=[
            jax.ShapeDtypeStruct((NN, 1), jnp.float32),
            jax.ShapeDtypeStruct((NN, FF), jnp.float32),
        ],
    )(hist, xw1)


def _tc_combine1(parts, xw1, dinv, b1r, w2):
    def body(p_ref, xw_ref, d_ref, b_ref, w_ref, xw2_ref, y2_ref):
        dinv = d_ref[...]
        h = dinv * (p_ref[0] + p_ref[1]) + dinv * dinv * xw_ref[...] + b_ref[...]
        h = jnp.maximum(h, 0.0)
        xw2 = jnp.dot(h, w_ref[...], preferred_element_type=jnp.float32)
        xw2_ref[...] = xw2
        y2_ref[...] = dinv * xw2

    return pl.pallas_call(
        body,
        grid=(NB,),
        in_specs=[
            pl.BlockSpec((NC, TCB, FF), lambda i: (0, i, 0)),
            pl.BlockSpec((TCB, FF), lambda i: (i, 0)),
            pl.BlockSpec((TCB, 1), lambda i: (i, 0)),
            pl.BlockSpec((1, FF), lambda i: (0, 0)),
            pl.BlockSpec((FF, FF), lambda i: (0, 0)),
        ],
        out_specs=[
            pl.BlockSpec((TCB, FF), lambda i: (i, 0)),
            pl.BlockSpec((TCB, FF), lambda i: (i, 0)),
        ],
        out_shape=[
            jax.ShapeDtypeStruct((NN, FF), jnp.float32),
            jax.ShapeDtypeStruct((NN, FF), jnp.float32),
        ],
    )(parts, xw1, dinv, b1r, w2)


def _tc_pool(parts, xw2, dinv, b2r, batch3d):
    def body(p_ref, xw_ref, d_ref, b_ref, bt_ref, sums_ref, cnts_ref):
        i = pl.program_id(0)
        dinv = d_ref[...]
        h2 = dinv * (p_ref[0] + p_ref[1]) + dinv * dinv * xw_ref[...] + b_ref[...]
        bt = bt_ref[0, 0, :]
        gids = lax.broadcasted_iota(jnp.int32, (1, NG), 1)
        onehot = (bt[:, None] == gids).astype(jnp.float32)
        contrib = lax.dot_general(onehot, h2, (((0,), (0,)), ((), ())),
                                  preferred_element_type=jnp.float32)
        cnt = lax.dot_general(onehot, jnp.ones((TCB, FF), jnp.float32),
                              (((0,), (0,)), ((), ())),
                              preferred_element_type=jnp.float32)

        @pl.when(i == 0)
        def _():
            sums_ref[...] = jnp.zeros_like(sums_ref)
            cnts_ref[...] = jnp.zeros_like(cnts_ref)

        sums_ref[...] += contrib
        cnts_ref[...] += cnt

    return pl.pallas_call(
        body,
        grid=(NB,),
        in_specs=[
            pl.BlockSpec((NC, TCB, FF), lambda i: (0, i, 0)),
            pl.BlockSpec((TCB, FF), lambda i: (i, 0)),
            pl.BlockSpec((TCB, 1), lambda i: (i, 0)),
            pl.BlockSpec((1, FF), lambda i: (0, 0)),
            pl.BlockSpec((1, 1, TCB), lambda i: (i, 0, 0)),
        ],
        out_specs=[
            pl.BlockSpec((NG, FF), lambda i: (0, 0)),
            pl.BlockSpec((NG, FF), lambda i: (0, 0)),
        ],
        out_shape=[
            jax.ShapeDtypeStruct((NG, FF), jnp.float32),
            jax.ShapeDtypeStruct((NG, FF), jnp.float32),
        ],
    )(parts, xw2, dinv, b2r, batch3d)


def _tc_final(sums, cnts, wm, bmr):
    def body(s_ref, c_ref, w_ref, b_ref, o_ref):
        pooled = s_ref[...] / jnp.maximum(c_ref[...], 1.0)
        o_ref[...] = jnp.dot(pooled, w_ref[...],
                             preferred_element_type=jnp.float32) + b_ref[...]

    return pl.pallas_call(
        body,
        out_shape=jax.ShapeDtypeStruct((NG, FF), jnp.float32),
    )(sums, cnts, wm, bmr)


def kernel(x, adj, batch, W1, b1, W2, b2, Wm, bm):
    src3d = adj[0].reshape(WORKERS, CPW, CH)
    dst3d = adj[1].reshape(WORKERS, CPW, CH)
    dst_flat = adj[1]
    batch3d = batch.reshape(NB, 1, TCB)
    b1r = b1.reshape(1, FF)
    b2r = b2.reshape(1, FF)
    bmr = bm.reshape(1, FF)

    hist = _sc_deg(dst_flat)
    xw1 = _tc_matmul(x, W1)
    hist3d = hist.reshape(WORKERS, NB, TCB).transpose(1, 0, 2)
    dinv, y1 = _tc_prep(hist3d, xw1)
    p1 = _sc_agg(y1, src3d, dst3d)
    xw2, y2 = _tc_combine1(p1, xw1, dinv, b1r, W2)
    p2 = _sc_agg(y2, src3d, dst3d)
    sums, cnts = _tc_pool(p2, xw2, dinv, b2r, batch3d)
    return _tc_final(sums, cnts, Wm, bmr)


# speedup vs baseline: 25.3678x; 1.2567x over previous
import dataclasses
import functools

import jax
import jax.numpy as jnp
from jax import lax
from jax.experimental import pallas as pl
from jax.experimental.pallas import tpu as pltpu
from jax.experimental.pallas import tpu_sc as plsc

NN = 10000
EE = 320000
FF = 128
NG = 64
NC, NS = 2, 16
WORKERS = NC * NS
CH = 80
NCHUNK = EE // CH
CPW = NCHUNK // WORKERS
EPW = EE // WORKERS
NP = 10240
RPS = NP // NS
RB = CH
TCB = 1000
NB = NN // TCB


def _vmesh():
    return plsc.VectorSubcoreMesh(core_axis_name="c", subcore_axis_name="s")


def _sc_params():
    cp = pltpu.CompilerParams()
    if "needs_layout_passes" in pltpu.CompilerParams.__dataclass_fields__:
        cp = dataclasses.replace(cp, needs_layout_passes=False)
    return cp


def _sc_deg(dst_flat):
    @functools.partial(
        pl.kernel,
        out_type=jax.ShapeDtypeStruct((WORKERS, NN), jnp.float32),
        mesh=_vmesh(),
        compiler_params=_sc_params(),
        scratch_types=[
            pltpu.VMEM((EPW,), jnp.int32),
            pltpu.VMEM((NN,), jnp.float32),
        ],
    )
    def k(dst_hbm, out_hbm, idx_v, hist_v):
        c = lax.axis_index("c")
        s = lax.axis_index("s")
        wid = s * NC + c

        @pl.loop(0, NN // 16)
        def _(i):
            hist_v[pl.ds(i * 16, 16)] = jnp.zeros((16,), jnp.float32)

        pltpu.sync_copy(dst_hbm.at[pl.ds(wid * EPW, EPW)], idx_v)
        ones = jnp.ones((16,), jnp.float32)

        @pl.loop(0, EPW // 16)
        def _(i):
            idx = idx_v[pl.ds(i * 16, 16)]
            plsc.addupdate_scatter(hist_v, [idx], ones)

        pltpu.sync_copy(hist_v, out_hbm.at[wid])

    return k(dst_flat)


def _sc_agg(y, src_flat, dst3d):
    @functools.partial(
        pl.kernel,
        out_type=jax.ShapeDtypeStruct((NC, NP, FF), jnp.float32),
        mesh=_vmesh(),
        scratch_types=[
            pltpu.VMEM((EPW,), jnp.int32),
            pltpu.VMEM((CPW, CH), jnp.int32),
            pltpu.VMEM((CH, FF), jnp.float32),
            pltpu.VMEM((CH, FF), jnp.float32),
            pltpu.VMEM_SHARED((NP, FF), jnp.float32),
            pltpu.SemaphoreType.DMA,
            pltpu.SemaphoreType.DMA,
        ],
    )
    def k(y_hbm, src_hbm, dst_hbm, out_hbm, src_v, dst_v, g0, g1, acc,
          ss0, ss1):
        c = lax.axis_index("c")
        s = lax.axis_index("s")
        wid = s * NC + c

        @pl.loop(0, RB)
        def _(r):
            @pl.loop(0, FF // 16)
            def _(q):
                g0[r, pl.ds(q * 16, 16)] = jnp.zeros((16,), jnp.float32)

        @pl.loop(0, RPS // RB)
        def _(t):
            pltpu.sync_copy(g0, acc.at[pl.ds(s * RPS + t * RB, RB)])

        plsc.subcore_barrier()

        pltpu.sync_copy(src_hbm.at[pl.ds(wid * EPW, EPW)], src_v)
        pltpu.sync_copy(dst_hbm.at[wid], dst_v)

        def _gather(j, buf):
            pltpu.sync_copy(y_hbm.at[src_v.at[pl.ds(j * CH, CH)]], buf)

        _gather(0, g0)
        pltpu.async_copy(g0, acc.at[dst_v.at[0]], ss0, add=True)
        _gather(1, g1)
        pltpu.async_copy(g1, acc.at[dst_v.at[1]], ss1, add=True)

        @pl.loop(1, (CPW - 1) // 2)
        def _(t):
            j = 2 * t
            pltpu.make_async_copy(g0, acc.at[dst_v.at[j]], ss0).wait()
            _gather(j, g0)
            pltpu.async_copy(g0, acc.at[dst_v.at[j]], ss0, add=True)
            pltpu.make_async_copy(g1, acc.at[dst_v.at[j + 1]], ss1).wait()
            _gather(j + 1, g1)
            pltpu.async_copy(g1, acc.at[dst_v.at[j + 1]], ss1, add=True)

        pltpu.make_async_copy(g0, acc.at[dst_v.at[CPW - 1]], ss0).wait()
        _gather(CPW - 1, g0)
        pltpu.async_copy(g0, acc.at[dst_v.at[CPW - 1]], ss0, add=True)

        pltpu.make_async_copy(g0, acc.at[dst_v.at[CPW - 1]], ss0).wait()
        pltpu.make_async_copy(g1, acc.at[dst_v.at[CPW - 2]], ss1).wait()

        plsc.subcore_barrier()

        @pl.loop(0, RPS // RB)
        def _(t):
            pltpu.sync_copy(
                acc.at[pl.ds(s * RPS + t * RB, RB)],
                out_hbm.at[c, pl.ds(s * RPS + t * RB, RB)],
            )

    return k(y, src_flat, dst3d)


def _tc_matmul(x, w):
    def body(x_ref, w_ref, o_ref):
        o_ref[...] = jnp.dot(x_ref[...], w_ref[...],
                             preferred_element_type=jnp.float32)

    return pl.pallas_call(
        body,
        grid=(NB,),
        in_specs=[
            pl.BlockSpec((TCB, FF), lambda i: (i, 0)),
            pl.BlockSpec((FF, FF), lambda i: (0, 0)),
        ],
        out_specs=pl.BlockSpec((TCB, FF), lambda i: (i, 0)),
        out_shape=jax.ShapeDtypeStruct((NN, FF), jnp.float32),
    )(x, w)


def _tc_prep(hist, xw1):
    def body(h_ref, xw_ref, dinv_ref, y_ref):
        deg = jnp.sum(h_ref[0], axis=0) + 1.0
        dinv = lax.rsqrt(deg)
        dinv_ref[...] = dinv[:, None]
        y_ref[...] = dinv[:, None] * xw_ref[...]

    return pl.pallas_call(
        body,
        grid=(NB,),
        in_specs=[
            pl.BlockSpec((1, WORKERS, TCB), lambda i: (i, 0, 0)),
            pl.BlockSpec((TCB, FF), lambda i: (i, 0)),
        ],
        out_specs=[
            pl.BlockSpec((TCB, 1), lambda i: (i, 0)),
            pl.BlockSpec((TCB, FF), lambda i: (i, 0)),
        ],
        out_shape=[
            jax.ShapeDtypeStruct((NN, 1), jnp.float32),
            jax.ShapeDtypeStruct((NN, FF), jnp.float32),
        ],
    )(hist, xw1)


def _tc_combine1(parts, xw1, dinv, b1r, w2):
    def body(p_ref, xw_ref, d_ref, b_ref, w_ref, xw2_ref, y2_ref):
        dinv = d_ref[...]
        h = dinv * (p_ref[0] + p_ref[1]) + dinv * dinv * xw_ref[...] + b_ref[...]
        h = jnp.maximum(h, 0.0)
        xw2 = jnp.dot(h, w_ref[...], preferred_element_type=jnp.float32)
        xw2_ref[...] = xw2
        y2_ref[...] = dinv * xw2

    return pl.pallas_call(
        body,
        grid=(NB,),
        in_specs=[
            pl.BlockSpec((NC, TCB, FF), lambda i: (0, i, 0)),
            pl.BlockSpec((TCB, FF), lambda i: (i, 0)),
            pl.BlockSpec((TCB, 1), lambda i: (i, 0)),
            pl.BlockSpec((1, FF), lambda i: (0, 0)),
            pl.BlockSpec((FF, FF), lambda i: (0, 0)),
        ],
        out_specs=[
            pl.BlockSpec((TCB, FF), lambda i: (i, 0)),
            pl.BlockSpec((TCB, FF), lambda i: (i, 0)),
        ],
        out_shape=[
            jax.ShapeDtypeStruct((NN, FF), jnp.float32),
            jax.ShapeDtypeStruct((NN, FF), jnp.float32),
        ],
    )(parts, xw1, dinv, b1r, w2)


def _tc_pool(parts, xw2, dinv, b2r, batch3d):
    def body(p_ref, xw_ref, d_ref, b_ref, bt_ref, sums_ref, cnts_ref):
        i = pl.program_id(0)
        dinv = d_ref[...]
        h2 = dinv * (p_ref[0] + p_ref[1]) + dinv * dinv * xw_ref[...] + b_ref[...]
        bt = bt_ref[0, 0, :]
        gids = lax.broadcasted_iota(jnp.int32, (1, NG), 1)
        onehot = (bt[:, None] == gids).astype(jnp.float32)
        contrib = lax.dot_general(onehot, h2, (((0,), (0,)), ((), ())),
                                  preferred_element_type=jnp.float32)
        cnt = lax.dot_general(onehot, jnp.ones((TCB, FF), jnp.float32),
                              (((0,), (0,)), ((), ())),
                              preferred_element_type=jnp.float32)

        @pl.when(i == 0)
        def _():
            sums_ref[...] = jnp.zeros_like(sums_ref)
            cnts_ref[...] = jnp.zeros_like(cnts_ref)

        sums_ref[...] += contrib
        cnts_ref[...] += cnt

    return pl.pallas_call(
        body,
        grid=(NB,),
        in_specs=[
            pl.BlockSpec((NC, TCB, FF), lambda i: (0, i, 0)),
            pl.BlockSpec((TCB, FF), lambda i: (i, 0)),
            pl.BlockSpec((TCB, 1), lambda i: (i, 0)),
            pl.BlockSpec((1, FF), lambda i: (0, 0)),
            pl.BlockSpec((1, 1, TCB), lambda i: (i, 0, 0)),
        ],
        out_specs=[
            pl.BlockSpec((NG, FF), lambda i: (0, 0)),
            pl.BlockSpec((NG, FF), lambda i: (0, 0)),
        ],
        out_shape=[
            jax.ShapeDtypeStruct((NG, FF), jnp.float32),
            jax.ShapeDtypeStruct((NG, FF), jnp.float32),
        ],
    )(parts, xw2, dinv, b2r, batch3d)


def _tc_final(sums, cnts, wm, bmr):
    def body(s_ref, c_ref, w_ref, b_ref, o_ref):
        pooled = s_ref[...] / jnp.maximum(c_ref[...], 1.0)
        o_ref[...] = jnp.dot(pooled, w_ref[...],
                             preferred_element_type=jnp.float32) + b_ref[...]

    return pl.pallas_call(
        body,
        out_shape=jax.ShapeDtypeStruct((NG, FF), jnp.float32),
    )(sums, cnts, wm, bmr)


def kernel(x, adj, batch, W1, b1, W2, b2, Wm, bm):
    src_flat = adj[0]
    dst3d = adj[1].reshape(WORKERS, CPW, CH)
    dst_flat = adj[1]
    batch3d = batch.reshape(NB, 1, TCB)
    b1r = b1.reshape(1, FF)
    b2r = b2.reshape(1, FF)
    bmr = bm.reshape(1, FF)

    hist = _sc_deg(dst_flat)
    xw1 = _tc_matmul(x, W1)
    hist3d = hist.reshape(WORKERS, NB, TCB).transpose(1, 0, 2)
    dinv, y1 = _tc_prep(hist3d, xw1)
    p1 = _sc_agg(y1, src_flat, dst3d)
    xw2, y2 = _tc_combine1(p1, xw1, dinv, b1r, W2)
    p2 = _sc_agg(y2, src_flat, dst3d)
    sums, cnts = _tc_pool(p2, xw2, dinv, b2r, batch3d)
    return _tc_final(sums, cnts, Wm, bmr)


# speedup vs baseline: 27.4480x; 1.0820x over previous
import dataclasses
import functools

import jax
import jax.numpy as jnp
from jax import lax
from jax.experimental import pallas as pl
from jax.experimental.pallas import tpu as pltpu
from jax.experimental.pallas import tpu_sc as plsc

NN = 10000
EE = 320000
FF = 128
NG = 64
NC, NS = 2, 16
WORKERS = NC * NS
CH = 40
DEPTH = 5
NCHUNK = EE // CH
CPW = NCHUNK // WORKERS
EPW = EE // WORKERS
NP = 10240
RPS = NP // NS
RB = CH
TCB = 1000
NB = NN // TCB


def _vmesh():
    return plsc.VectorSubcoreMesh(core_axis_name="c", subcore_axis_name="s")


def _sc_params():
    cp = pltpu.CompilerParams()
    if "needs_layout_passes" in pltpu.CompilerParams.__dataclass_fields__:
        cp = dataclasses.replace(cp, needs_layout_passes=False)
    return cp


def _sc_deg(dst_flat):
    @functools.partial(
        pl.kernel,
        out_type=jax.ShapeDtypeStruct((WORKERS, NN), jnp.float32),
        mesh=_vmesh(),
        compiler_params=_sc_params(),
        scratch_types=[
            pltpu.VMEM((EPW,), jnp.int32),
            pltpu.VMEM((NN,), jnp.float32),
        ],
    )
    def k(dst_hbm, out_hbm, idx_v, hist_v):
        c = lax.axis_index("c")
        s = lax.axis_index("s")
        wid = s * NC + c

        @pl.loop(0, NN // 16)
        def _(i):
            hist_v[pl.ds(i * 16, 16)] = jnp.zeros((16,), jnp.float32)

        pltpu.sync_copy(dst_hbm.at[pl.ds(wid * EPW, EPW)], idx_v)
        ones = jnp.ones((16,), jnp.float32)

        @pl.loop(0, EPW // 16)
        def _(i):
            idx = idx_v[pl.ds(i * 16, 16)]
            plsc.addupdate_scatter(hist_v, [idx], ones)

        pltpu.sync_copy(hist_v, out_hbm.at[wid])

    return k(dst_flat)


def _sc_agg(y, src_flat, dst_flat):
    @functools.partial(
        pl.kernel,
        out_type=jax.ShapeDtypeStruct((NC, NP, FF), jnp.float32),
        mesh=_vmesh(),
        scratch_types=(
            [
                pltpu.VMEM((EPW,), jnp.int32),
                pltpu.VMEM((EPW,), jnp.int32),
            ]
            + [pltpu.VMEM((CH, FF), jnp.float32) for _ in range(DEPTH)]
            + [pltpu.SemaphoreType.DMA for _ in range(2 * DEPTH)]
            + [pltpu.VMEM_SHARED((NP, FF), jnp.float32)]
        ),
    )
    def k(y_hbm, src_hbm, dst_hbm, out_hbm, src_v, dst_v, *rest):
        bufs = rest[:DEPTH]
        sg = rest[DEPTH:2 * DEPTH]
        ss = rest[2 * DEPTH:3 * DEPTH]
        acc = rest[3 * DEPTH]
        c = lax.axis_index("c")
        s = lax.axis_index("s")
        wid = s * NC + c

        g0 = bufs[0]

        @pl.loop(0, RB)
        def _(r):
            @pl.loop(0, FF // 16)
            def _(q):
                g0[r, pl.ds(q * 16, 16)] = jnp.zeros((16,), jnp.float32)

        @pl.loop(0, RPS // RB)
        def _(t):
            pltpu.sync_copy(g0, acc.at[pl.ds(s * RPS + t * RB, RB)])

        plsc.subcore_barrier()

        pltpu.sync_copy(src_hbm.at[pl.ds(wid * EPW, EPW)], src_v)
        pltpu.sync_copy(dst_hbm.at[pl.ds(wid * EPW, EPW)], dst_v)

        def _src_at(j):
            return src_v.at[pl.ds(j * CH, CH)]

        def _dst_at(j):
            return dst_v.at[pl.ds(j * CH, CH)]

        def _step(j, b, refill, wait_prev):
            pltpu.make_async_copy(y_hbm.at[_src_at(j)], bufs[b], sg[b]).wait()
            pltpu.async_copy(bufs[b], acc.at[_dst_at(j)], ss[b], add=True)
            if refill:
                b2 = (b + 2) % DEPTH
                if wait_prev:
                    pltpu.make_async_copy(bufs[b2], acc.at[_dst_at(j)],
                                          ss[b2]).wait()
                pltpu.async_copy(y_hbm.at[_src_at(j + 2)], bufs[b2], sg[b2])

        pltpu.async_copy(y_hbm.at[_src_at(0)], bufs[0], sg[0])
        pltpu.async_copy(y_hbm.at[_src_at(1)], bufs[1], sg[1])
        for j in range(DEPTH - 2):
            _step(j, j, True, False)

        GROUPS = (CPW - 2 * (DEPTH - 2)) // DEPTH
        @pl.loop(0, GROUPS)
        def _(t):
            for i in range(DEPTH):
                _step(t * DEPTH + (DEPTH - 2) + i, (DEPTH - 2 + i) % DEPTH,
                      True, True)

        for j in range(GROUPS * DEPTH + DEPTH - 2, CPW):
            _step(j, j % DEPTH, j + 2 < CPW, True)

        for j in range(CPW - DEPTH, CPW):
            pltpu.make_async_copy(bufs[j % DEPTH], acc.at[_dst_at(0)],
                                  ss[j % DEPTH]).wait()

        plsc.subcore_barrier()

        @pl.loop(0, RPS // RB)
        def _(t):
            pltpu.sync_copy(
                acc.at[pl.ds(s * RPS + t * RB, RB)],
                out_hbm.at[c, pl.ds(s * RPS + t * RB, RB)],
            )

    return k(y, src_flat, dst_flat)


def _tc_matmul(x, w):
    def body(x_ref, w_ref, o_ref):
        o_ref[...] = jnp.dot(x_ref[...], w_ref[...],
                             preferred_element_type=jnp.float32)

    return pl.pallas_call(
        body,
        grid=(NB,),
        in_specs=[
            pl.BlockSpec((TCB, FF), lambda i: (i, 0)),
            pl.BlockSpec((FF, FF), lambda i: (0, 0)),
        ],
        out_specs=pl.BlockSpec((TCB, FF), lambda i: (i, 0)),
        out_shape=jax.ShapeDtypeStruct((NN, FF), jnp.float32),
    )(x, w)


def _tc_prep(hist, xw1):
    def body(h_ref, xw_ref, dinv_ref, y_ref):
        deg = jnp.sum(h_ref[0], axis=0) + 1.0
        dinv = lax.rsqrt(deg)
        dinv_ref[...] = dinv[:, None]
        y_ref[...] = dinv[:, None] * xw_ref[...]

    return pl.pallas_call(
        body,
        grid=(NB,),
        in_specs=[
            pl.BlockSpec((1, WORKERS, TCB), lambda i: (i, 0, 0)),
            pl.BlockSpec((TCB, FF), lambda i: (i, 0)),
        ],
        out_specs=[
            pl.BlockSpec((TCB, 1), lambda i: (i, 0)),
            pl.BlockSpec((TCB, FF), lambda i: (i, 0)),
        ],
        out_shape=[
            jax.ShapeDtypeStruct((NN, 1), jnp.float32),
            jax.ShapeDtypeStruct((NN, FF), jnp.float32),
        ],
    )(hist, xw1)


def _tc_combine1(parts, xw1, dinv, b1r, w2):
    def body(p_ref, xw_ref, d_ref, b_ref, w_ref, xw2_ref, y2_ref):
        dinv = d_ref[...]
        h = dinv * (p_ref[0] + p_ref[1]) + dinv * dinv * xw_ref[...] + b_ref[...]
        h = jnp.maximum(h, 0.0)
        xw2 = jnp.dot(h, w_ref[...], preferred_element_type=jnp.float32)
        xw2_ref[...] = xw2
        y2_ref[...] = dinv * xw2

    return pl.pallas_call(
        body,
        grid=(NB,),
        in_specs=[
            pl.BlockSpec((NC, TCB, FF), lambda i: (0, i, 0)),
            pl.BlockSpec((TCB, FF), lambda i: (i, 0)),
            pl.BlockSpec((TCB, 1), lambda i: (i, 0)),
            pl.BlockSpec((1, FF), lambda i: (0, 0)),
            pl.BlockSpec((FF, FF), lambda i: (0, 0)),
        ],
        out_specs=[
            pl.BlockSpec((TCB, FF), lambda i: (i, 0)),
            pl.BlockSpec((TCB, FF), lambda i: (i, 0)),
        ],
        out_shape=[
            jax.ShapeDtypeStruct((NN, FF), jnp.float32),
            jax.ShapeDtypeStruct((NN, FF), jnp.float32),
        ],
    )(parts, xw1, dinv, b1r, w2)


def _tc_pool(parts, xw2, dinv, b2r, batch3d):
    def body(p_ref, xw_ref, d_ref, b_ref, bt_ref, sums_ref, cnts_ref):
        i = pl.program_id(0)
        dinv = d_ref[...]
        h2 = dinv * (p_ref[0] + p_ref[1]) + dinv * dinv * xw_ref[...] + b_ref[...]
        bt = bt_ref[0, 0, :]
        gids = lax.broadcasted_iota(jnp.int32, (1, NG), 1)
        onehot = (bt[:, None] == gids).astype(jnp.float32)
        contrib = lax.dot_general(onehot, h2, (((0,), (0,)), ((), ())),
                                  preferred_element_type=jnp.float32)
        cnt = lax.dot_general(onehot, jnp.ones((TCB, FF), jnp.float32),
                              (((0,), (0,)), ((), ())),
                              preferred_element_type=jnp.float32)

        @pl.when(i == 0)
        def _():
            sums_ref[...] = jnp.zeros_like(sums_ref)
            cnts_ref[...] = jnp.zeros_like(cnts_ref)

        sums_ref[...] += contrib
        cnts_ref[...] += cnt

    return pl.pallas_call(
        body,
        grid=(NB,),
        in_specs=[
            pl.BlockSpec((NC, TCB, FF), lambda i: (0, i, 0)),
            pl.BlockSpec((TCB, FF), lambda i: (i, 0)),
            pl.BlockSpec((TCB, 1), lambda i: (i, 0)),
            pl.BlockSpec((1, FF), lambda i: (0, 0)),
            pl.BlockSpec((1, 1, TCB), lambda i: (i, 0, 0)),
        ],
        out_specs=[
            pl.BlockSpec((NG, FF), lambda i: (0, 0)),
            pl.BlockSpec((NG, FF), lambda i: (0, 0)),
        ],
        out_shape=[
            jax.ShapeDtypeStruct((NG, FF), jnp.float32),
            jax.ShapeDtypeStruct((NG, FF), jnp.float32),
        ],
    )(parts, xw2, dinv, b2r, batch3d)


def _tc_final(sums, cnts, wm, bmr):
    def body(s_ref, c_ref, w_ref, b_ref, o_ref):
        pooled = s_ref[...] / jnp.maximum(c_ref[...], 1.0)
        o_ref[...] = jnp.dot(pooled, w_ref[...],
                             preferred_element_type=jnp.float32) + b_ref[...]

    return pl.pallas_call(
        body,
        out_shape=jax.ShapeDtypeStruct((NG, FF), jnp.float32),
    )(sums, cnts, wm, bmr)


def kernel(x, adj, batch, W1, b1, W2, b2, Wm, bm):
    src_flat = adj[0]
    dst_flat = adj[1]
    batch3d = batch.reshape(NB, 1, TCB)
    b1r = b1.reshape(1, FF)
    b2r = b2.reshape(1, FF)
    bmr = bm.reshape(1, FF)

    hist = _sc_deg(dst_flat)
    xw1 = _tc_matmul(x, W1)
    hist3d = hist.reshape(WORKERS, NB, TCB).transpose(1, 0, 2)
    dinv, y1 = _tc_prep(hist3d, xw1)
    p1 = _sc_agg(y1, src_flat, dst_flat)
    xw2, y2 = _tc_combine1(p1, xw1, dinv, b1r, W2)
    p2 = _sc_agg(y2, src_flat, dst_flat)
    sums, cnts = _tc_pool(p2, xw2, dinv, b2r, batch3d)
    return _tc_final(sums, cnts, Wm, bmr)


# speedup vs baseline: 35.9357x; 1.3092x over previous
import dataclasses
import functools

import jax
import jax.numpy as jnp
from jax import lax
from jax.experimental import pallas as pl
from jax.experimental.pallas import tpu as pltpu
from jax.experimental.pallas import tpu_sc as plsc

NN = 10000
EE = 320000
FF = 128
NG = 64
NC, NS = 2, 16
WORKERS = NC * NS
CH = 80
DEPTH = 3
NCHUNK = EE // CH
CPW = NCHUNK // WORKERS
EPW = EE // WORKERS
RB = CH
TCB = 1000
NB = NN // TCB


def _vmesh():
    return plsc.VectorSubcoreMesh(core_axis_name="c", subcore_axis_name="s")


def _sc_params():
    cp = pltpu.CompilerParams()
    if "needs_layout_passes" in pltpu.CompilerParams.__dataclass_fields__:
        cp = dataclasses.replace(cp, needs_layout_passes=False)
    return cp


def _sc_deg(dst_flat):
    @functools.partial(
        pl.kernel,
        out_type=jax.ShapeDtypeStruct((WORKERS, NN), jnp.float32),
        mesh=_vmesh(),
        compiler_params=_sc_params(),
        scratch_types=[
            pltpu.VMEM((EPW,), jnp.int32),
            pltpu.VMEM((NN,), jnp.float32),
        ],
    )
    def k(dst_hbm, out_hbm, idx_v, hist_v):
        c = lax.axis_index("c")
        s = lax.axis_index("s")
        wid = s * NC + c

        @pl.loop(0, NN // 16)
        def _(i):
            hist_v[pl.ds(i * 16, 16)] = jnp.zeros((16,), jnp.float32)

        pltpu.sync_copy(dst_hbm.at[pl.ds(wid * EPW, EPW)], idx_v)
        ones = jnp.ones((16,), jnp.float32)

        @pl.loop(0, EPW // 16)
        def _(i):
            idx = idx_v[pl.ds(i * 16, 16)]
            plsc.addupdate_scatter(hist_v, [idx], ones)

        pltpu.sync_copy(hist_v, out_hbm.at[wid])

    return k(dst_flat)


def _sc_agg(y, src_flat, dst_flat):
    @functools.partial(
        pl.kernel,
        out_type=jax.ShapeDtypeStruct((NC, NN, FF), jnp.float32),
        mesh=_vmesh(),
        scratch_types=(
            [
                pltpu.VMEM((EPW,), jnp.int32),
                pltpu.VMEM((EPW,), jnp.int32),
            ]
            + [pltpu.VMEM((CH, FF), jnp.float32) for _ in range(DEPTH)]
            + [pltpu.SemaphoreType.DMA for _ in range(2 * DEPTH)]
            + [pltpu.VMEM_SHARED((NN, FF), jnp.float32)]
        ),
    )
    def k(y_hbm, src_hbm, dst_hbm, out_hbm, src_v, dst_v, *rest):
        bufs = rest[:DEPTH]
        sg = rest[DEPTH:2 * DEPTH]
        ss = rest[2 * DEPTH:3 * DEPTH]
        acc = rest[3 * DEPTH]
        c = lax.axis_index("c")
        s = lax.axis_index("s")
        wid = s * NC + c
        nz = jnp.where(s < 13, 8, 7)
        base = 640 * s - 80 * jnp.maximum(s - 13, 0)

        g0 = bufs[0]

        @pl.loop(0, RB)
        def _(r):
            @pl.loop(0, FF // 16)
            def _(q):
                g0[r, pl.ds(q * 16, 16)] = jnp.zeros((16,), jnp.float32)

        @pl.loop(0, 8)
        def _(t):
            @pl.when(t < nz)
            def _():
                pltpu.sync_copy(g0, acc.at[pl.ds(base + t * RB, RB)])

        plsc.subcore_barrier()

        pltpu.sync_copy(src_hbm.at[pl.ds(wid * EPW, EPW)], src_v)
        pltpu.sync_copy(dst_hbm.at[pl.ds(wid * EPW, EPW)], dst_v)

        def _src_at(j):
            return src_v.at[pl.ds(j * CH, CH)]

        def _dst_at(j):
            return dst_v.at[pl.ds(j * CH, CH)]

        def _step(j, b, refill, wait_prev):
            pltpu.make_async_copy(y_hbm.at[_src_at(j)], bufs[b], sg[b]).wait()
            pltpu.async_copy(bufs[b], acc.at[_dst_at(j)], ss[b], add=True)
            if refill:
                b2 = (b + 2) % DEPTH
                if wait_prev:
                    pltpu.make_async_copy(bufs[b2], acc.at[_dst_at(j)],
                                          ss[b2]).wait()
                pltpu.async_copy(y_hbm.at[_src_at(j + 2)], bufs[b2], sg[b2])

        pltpu.async_copy(y_hbm.at[_src_at(0)], bufs[0], sg[0])
        pltpu.async_copy(y_hbm.at[_src_at(1)], bufs[1], sg[1])
        for j in range(DEPTH - 2):
            _step(j, j, True, False)

        GROUPS = (CPW - DEPTH) // DEPTH
        @pl.loop(0, GROUPS)
        def _(t):
            for i in range(DEPTH):
                _step(t * DEPTH + (DEPTH - 2) + i, (DEPTH - 2 + i) % DEPTH,
                      True, True)

        for j in range(GROUPS * DEPTH + DEPTH - 2, CPW):
            _step(j, j % DEPTH, j + 2 < CPW, True)

        for j in range(CPW - DEPTH, CPW):
            pltpu.make_async_copy(bufs[j % DEPTH], acc.at[_dst_at(0)],
                                  ss[j % DEPTH]).wait()

        plsc.subcore_barrier()

        @pl.loop(0, 8)
        def _(t):
            @pl.when(t < nz)
            def _():
                pltpu.sync_copy(
                    acc.at[pl.ds(base + t * RB, RB)],
                    out_hbm.at[c, pl.ds(base + t * RB, RB)],
                )

    return k(y, src_flat, dst_flat)


def _tc_matmul(x, w):
    def body(x_ref, w_ref, o_ref):
        o_ref[...] = jnp.dot(x_ref[...], w_ref[...],
                             preferred_element_type=jnp.float32)

    return pl.pallas_call(
        body,
        grid=(NB,),
        in_specs=[
            pl.BlockSpec((TCB, FF), lambda i: (i, 0)),
            pl.BlockSpec((FF, FF), lambda i: (0, 0)),
        ],
        out_specs=pl.BlockSpec((TCB, FF), lambda i: (i, 0)),
        out_shape=jax.ShapeDtypeStruct((NN, FF), jnp.float32),
    )(x, w)


def _tc_prep(hist, xw1):
    def body(h_ref, xw_ref, dinv_ref, y_ref):
        deg = jnp.sum(h_ref[0], axis=0) + 1.0
        dinv = lax.rsqrt(deg)
        dinv_ref[...] = dinv[:, None]
        y_ref[...] = dinv[:, None] * xw_ref[...]

    return pl.pallas_call(
        body,
        grid=(NB,),
        in_specs=[
            pl.BlockSpec((1, WORKERS, TCB), lambda i: (i, 0, 0)),
            pl.BlockSpec((TCB, FF), lambda i: (i, 0)),
        ],
        out_specs=[
            pl.BlockSpec((TCB, 1), lambda i: (i, 0)),
            pl.BlockSpec((TCB, FF), lambda i: (i, 0)),
        ],
        out_shape=[
            jax.ShapeDtypeStruct((NN, 1), jnp.float32),
            jax.ShapeDtypeStruct((NN, FF), jnp.float32),
        ],
    )(hist, xw1)


def _tc_combine1(parts, xw1, dinv, b1r, w2):
    def body(p_ref, xw_ref, d_ref, b_ref, w_ref, xw2_ref, y2_ref):
        dinv = d_ref[...]
        h = dinv * (p_ref[0] + p_ref[1]) + dinv * dinv * xw_ref[...] + b_ref[...]
        h = jnp.maximum(h, 0.0)
        xw2 = jnp.dot(h, w_ref[...], preferred_element_type=jnp.float32)
        xw2_ref[...] = xw2
        y2_ref[...] = dinv * xw2

    return pl.pallas_call(
        body,
        grid=(NB,),
        in_specs=[
            pl.BlockSpec((NC, TCB, FF), lambda i: (0, i, 0)),
            pl.BlockSpec((TCB, FF), lambda i: (i, 0)),
            pl.BlockSpec((TCB, 1), lambda i: (i, 0)),
            pl.BlockSpec((1, FF), lambda i: (0, 0)),
            pl.BlockSpec((FF, FF), lambda i: (0, 0)),
        ],
        out_specs=[
            pl.BlockSpec((TCB, FF), lambda i: (i, 0)),
            pl.BlockSpec((TCB, FF), lambda i: (i, 0)),
        ],
        out_shape=[
            jax.ShapeDtypeStruct((NN, FF), jnp.float32),
            jax.ShapeDtypeStruct((NN, FF), jnp.float32),
        ],
    )(parts, xw1, dinv, b1r, w2)


def _tc_pool(parts, xw2, dinv, b2r, batch3d):
    def body(p_ref, xw_ref, d_ref, b_ref, bt_ref, sums_ref, cnts_ref):
        i = pl.program_id(0)
        dinv = d_ref[...]
        h2 = dinv * (p_ref[0] + p_ref[1]) + dinv * dinv * xw_ref[...] + b_ref[...]
        bt = bt_ref[0, 0, :]
        gids = lax.broadcasted_iota(jnp.int32, (1, NG), 1)
        onehot = (bt[:, None] == gids).astype(jnp.float32)
        contrib = lax.dot_general(onehot, h2, (((0,), (0,)), ((), ())),
                                  preferred_element_type=jnp.float32)
        cnt = lax.dot_general(onehot, jnp.ones((TCB, FF), jnp.float32),
                              (((0,), (0,)), ((), ())),
                              preferred_element_type=jnp.float32)

        @pl.when(i == 0)
        def _():
            sums_ref[...] = jnp.zeros_like(sums_ref)
            cnts_ref[...] = jnp.zeros_like(cnts_ref)

        sums_ref[...] += contrib
        cnts_ref[...] += cnt

    return pl.pallas_call(
        body,
        grid=(NB,),
        in_specs=[
            pl.BlockSpec((NC, TCB, FF), lambda i: (0, i, 0)),
            pl.BlockSpec((TCB, FF), lambda i: (i, 0)),
            pl.BlockSpec((TCB, 1), lambda i: (i, 0)),
            pl.BlockSpec((1, FF), lambda i: (0, 0)),
            pl.BlockSpec((1, 1, TCB), lambda i: (i, 0, 0)),
        ],
        out_specs=[
            pl.BlockSpec((NG, FF), lambda i: (0, 0)),
            pl.BlockSpec((NG, FF), lambda i: (0, 0)),
        ],
        out_shape=[
            jax.ShapeDtypeStruct((NG, FF), jnp.float32),
            jax.ShapeDtypeStruct((NG, FF), jnp.float32),
        ],
    )(parts, xw2, dinv, b2r, batch3d)


def _tc_final(sums, cnts, wm, bmr):
    def body(s_ref, c_ref, w_ref, b_ref, o_ref):
        pooled = s_ref[...] / jnp.maximum(c_ref[...], 1.0)
        o_ref[...] = jnp.dot(pooled, w_ref[...],
                             preferred_element_type=jnp.float32) + b_ref[...]

    return pl.pallas_call(
        body,
        out_shape=jax.ShapeDtypeStruct((NG, FF), jnp.float32),
    )(sums, cnts, wm, bmr)


def kernel(x, adj, batch, W1, b1, W2, b2, Wm, bm):
    src_flat = adj[0]
    dst_flat = adj[1]
    batch3d = batch.reshape(NB, 1, TCB)
    b1r = b1.reshape(1, FF)
    b2r = b2.reshape(1, FF)
    bmr = bm.reshape(1, FF)

    hist = _sc_deg(dst_flat)
    xw1 = _tc_matmul(x, W1)
    hist3d = hist.reshape(WORKERS, NB, TCB).transpose(1, 0, 2)
    dinv, y1 = _tc_prep(hist3d, xw1)
    p1 = _sc_agg(y1, src_flat, dst_flat)
    xw2, y2 = _tc_combine1(p1, xw1, dinv, b1r, W2)
    p2 = _sc_agg(y2, src_flat, dst_flat)
    sums, cnts = _tc_pool(p2, xw2, dinv, b2r, batch3d)
    return _tc_final(sums, cnts, Wm, bmr)


# speedup vs baseline: 36.7038x; 1.0214x over previous
import dataclasses
import functools

import jax
import jax.numpy as jnp
from jax import lax
from jax.experimental import pallas as pl
from jax.experimental.pallas import tpu as pltpu
from jax.experimental.pallas import tpu_sc as plsc

NN = 10000
EE = 320000
FF = 128
NG = 64
NC, NS = 2, 16
WORKERS = NC * NS
CH = 80
DEPTH = 3
NCHUNK = EE // CH
CPW = NCHUNK // WORKERS
EPW = EE // WORKERS
RB = CH
TCB = 1000
NB = NN // TCB


def _vmesh():
    return plsc.VectorSubcoreMesh(core_axis_name="c", subcore_axis_name="s")


def _sc_params():
    cp = pltpu.CompilerParams()
    if "needs_layout_passes" in pltpu.CompilerParams.__dataclass_fields__:
        cp = dataclasses.replace(cp, needs_layout_passes=False)
    return cp


def _sc_deg(dst_flat):
    @functools.partial(
        pl.kernel,
        out_type=jax.ShapeDtypeStruct((WORKERS, NN), jnp.float32),
        mesh=_vmesh(),
        compiler_params=_sc_params(),
        scratch_types=[
            pltpu.VMEM((EPW,), jnp.int32),
            pltpu.VMEM((NN,), jnp.float32),
        ],
    )
    def k(dst_hbm, out_hbm, idx_v, hist_v):
        c = lax.axis_index("c")
        s = lax.axis_index("s")
        wid = s * NC + c

        @pl.loop(0, NN // 16)
        def _(i):
            hist_v[pl.ds(i * 16, 16)] = jnp.zeros((16,), jnp.float32)

        pltpu.sync_copy(dst_hbm.at[pl.ds(wid * EPW, EPW)], idx_v)
        ones = jnp.ones((16,), jnp.float32)

        @pl.loop(0, EPW // 16)
        def _(i):
            idx = idx_v[pl.ds(i * 16, 16)]
            plsc.addupdate_scatter(hist_v, [idx], ones)

        pltpu.sync_copy(hist_v, out_hbm.at[wid])

    return k(dst_flat)


def _sc_agg(y, src_flat, dst_flat):
    @functools.partial(
        pl.kernel,
        out_type=jax.ShapeDtypeStruct((NC, NN, FF), jnp.float32),
        mesh=_vmesh(),
        scratch_types=(
            [
                pltpu.VMEM((EPW,), jnp.int32),
                pltpu.VMEM((EPW,), jnp.int32),
            ]
            + [pltpu.VMEM((CH, FF), jnp.float32) for _ in range(DEPTH)]
            + [pltpu.SemaphoreType.DMA for _ in range(2 * DEPTH)]
            + [pltpu.VMEM_SHARED((NN, FF), jnp.float32)]
        ),
    )
    def k(y_hbm, src_hbm, dst_hbm, out_hbm, src_v, dst_v, *rest):
        bufs = rest[:DEPTH]
        sg = rest[DEPTH:2 * DEPTH]
        ss = rest[2 * DEPTH:3 * DEPTH]
        acc = rest[3 * DEPTH]
        c = lax.axis_index("c")
        s = lax.axis_index("s")
        wid = s * NC + c
        nz = jnp.where(s < 13, 8, 7)
        base = 640 * s - 80 * jnp.maximum(s - 13, 0)

        g0 = bufs[0]

        @pl.loop(0, RB)
        def _(r):
            @pl.loop(0, FF // 16)
            def _(q):
                g0[r, pl.ds(q * 16, 16)] = jnp.zeros((16,), jnp.float32)

        @pl.loop(0, 8)
        def _(t):
            @pl.when(t < nz)
            def _():
                pltpu.sync_copy(g0, acc.at[pl.ds(base + t * RB, RB)])

        plsc.subcore_barrier()

        pltpu.sync_copy(src_hbm.at[pl.ds(wid * EPW, EPW)], src_v)
        pltpu.sync_copy(dst_hbm.at[pl.ds(wid * EPW, EPW)], dst_v)

        def _src_at(j):
            return src_v.at[pl.ds(j * CH, CH)]

        def _dst_at(j):
            return dst_v.at[pl.ds(j * CH, CH)]

        def _step(j, b, refill, wait_prev):
            pltpu.make_async_copy(y_hbm.at[_src_at(j)], bufs[b], sg[b]).wait()
            pltpu.async_copy(bufs[b], acc.at[_dst_at(j)], ss[b], add=True)
            if refill:
                b2 = (b + 2) % DEPTH
                if wait_prev:
                    pltpu.make_async_copy(bufs[b2], acc.at[_dst_at(j)],
                                          ss[b2]).wait()
                pltpu.async_copy(y_hbm.at[_src_at(j + 2)], bufs[b2], sg[b2])

        pltpu.async_copy(y_hbm.at[_src_at(0)], bufs[0], sg[0])
        pltpu.async_copy(y_hbm.at[_src_at(1)], bufs[1], sg[1])
        for j in range(DEPTH - 2):
            _step(j, j, True, False)

        GROUPS = (CPW - DEPTH) // DEPTH
        @pl.loop(0, GROUPS)
        def _(t):
            for i in range(DEPTH):
                _step(t * DEPTH + (DEPTH - 2) + i, (DEPTH - 2 + i) % DEPTH,
                      True, True)

        for j in range(GROUPS * DEPTH + DEPTH - 2, CPW):
            _step(j, j % DEPTH, j + 2 < CPW, True)

        for j in range(CPW - DEPTH, CPW):
            pltpu.make_async_copy(bufs[j % DEPTH], acc.at[_dst_at(0)],
                                  ss[j % DEPTH]).wait()

        plsc.subcore_barrier()

        @pl.loop(0, 8)
        def _(t):
            @pl.when(t < nz)
            def _():
                pltpu.sync_copy(
                    acc.at[pl.ds(base + t * RB, RB)],
                    out_hbm.at[c, pl.ds(base + t * RB, RB)],
                )

    return k(y, src_flat, dst_flat)


def _tc_matmul(x, w):
    def body(x_ref, w_ref, o_ref):
        o_ref[...] = jnp.dot(x_ref[...], w_ref[...],
                             preferred_element_type=jnp.float32)

    return pl.pallas_call(
        body,
        grid=(NB,),
        in_specs=[
            pl.BlockSpec((TCB, FF), lambda i: (i, 0)),
            pl.BlockSpec((FF, FF), lambda i: (0, 0)),
        ],
        out_specs=pl.BlockSpec((TCB, FF), lambda i: (i, 0)),
        out_shape=jax.ShapeDtypeStruct((NN, FF), jnp.float32),
    )(x, w)


def _tc_prep(hist, xw1):
    PB = 1024

    def body(h_ref, xw_ref, dinv_ref, y_ref):
        deg = jnp.sum(h_ref[...], axis=0) + 1.0
        dinv = lax.rsqrt(deg)
        dinv_ref[...] = dinv[:, None]
        y_ref[...] = dinv[:, None] * xw_ref[...]

    return pl.pallas_call(
        body,
        grid=(NB,),
        in_specs=[
            pl.BlockSpec((WORKERS, PB), lambda i: (0, i)),
            pl.BlockSpec((PB, FF), lambda i: (i, 0)),
        ],
        out_specs=[
            pl.BlockSpec((PB, 1), lambda i: (i, 0)),
            pl.BlockSpec((PB, FF), lambda i: (i, 0)),
        ],
        out_shape=[
            jax.ShapeDtypeStruct((NN, 1), jnp.float32),
            jax.ShapeDtypeStruct((NN, FF), jnp.float32),
        ],
    )(hist, xw1)


def _tc_combine1(parts, xw1, dinv, b1r, w2):
    def body(p_ref, xw_ref, d_ref, b_ref, w_ref, xw2_ref, y2_ref):
        dinv = d_ref[...]
        h = dinv * (p_ref[0] + p_ref[1]) + dinv * dinv * xw_ref[...] + b_ref[...]
        h = jnp.maximum(h, 0.0)
        xw2 = jnp.dot(h, w_ref[...], preferred_element_type=jnp.float32)
        xw2_ref[...] = xw2
        y2_ref[...] = dinv * xw2

    return pl.pallas_call(
        body,
        grid=(NB,),
        in_specs=[
            pl.BlockSpec((NC, TCB, FF), lambda i: (0, i, 0)),
            pl.BlockSpec((TCB, FF), lambda i: (i, 0)),
            pl.BlockSpec((TCB, 1), lambda i: (i, 0)),
            pl.BlockSpec((1, FF), lambda i: (0, 0)),
            pl.BlockSpec((FF, FF), lambda i: (0, 0)),
        ],
        out_specs=[
            pl.BlockSpec((TCB, FF), lambda i: (i, 0)),
            pl.BlockSpec((TCB, FF), lambda i: (i, 0)),
        ],
        out_shape=[
            jax.ShapeDtypeStruct((NN, FF), jnp.float32),
            jax.ShapeDtypeStruct((NN, FF), jnp.float32),
        ],
    )(parts, xw1, dinv, b1r, w2)


def _tc_pool(parts, xw2, dinv, b2r, batch3d, wm, bmr):
    def body(p_ref, xw_ref, d_ref, b_ref, bt_ref, wm_ref, bm_ref,
             o_ref, sums_ref, cnts_ref):
        i = pl.program_id(0)
        dinv = d_ref[...]
        h2 = dinv * (p_ref[0] + p_ref[1]) + dinv * dinv * xw_ref[...] + b_ref[...]
        bt = bt_ref[0, 0, :]
        gids = lax.broadcasted_iota(jnp.int32, (1, NG), 1)
        onehot = (bt[:, None] == gids).astype(jnp.float32)
        contrib = lax.dot_general(onehot, h2, (((0,), (0,)), ((), ())),
                                  preferred_element_type=jnp.float32)
        cnt = lax.dot_general(onehot, jnp.ones((TCB, FF), jnp.float32),
                              (((0,), (0,)), ((), ())),
                              preferred_element_type=jnp.float32)

        @pl.when(i == 0)
        def _():
            sums_ref[...] = jnp.zeros_like(sums_ref)
            cnts_ref[...] = jnp.zeros_like(cnts_ref)

        sums_ref[...] += contrib
        cnts_ref[...] += cnt

        @pl.when(i == NB - 1)
        def _():
            pooled = sums_ref[...] / jnp.maximum(cnts_ref[...], 1.0)
            o_ref[...] = jnp.dot(pooled, wm_ref[...],
                                 preferred_element_type=jnp.float32) + bm_ref[...]

    out, _, _ = pl.pallas_call(
        body,
        grid=(NB,),
        in_specs=[
            pl.BlockSpec((NC, TCB, FF), lambda i: (0, i, 0)),
            pl.BlockSpec((TCB, FF), lambda i: (i, 0)),
            pl.BlockSpec((TCB, 1), lambda i: (i, 0)),
            pl.BlockSpec((1, FF), lambda i: (0, 0)),
            pl.BlockSpec((1, 1, TCB), lambda i: (i, 0, 0)),
            pl.BlockSpec((FF, FF), lambda i: (0, 0)),
            pl.BlockSpec((1, FF), lambda i: (0, 0)),
        ],
        out_specs=[
            pl.BlockSpec((NG, FF), lambda i: (0, 0)),
            pl.BlockSpec((NG, FF), lambda i: (0, 0)),
            pl.BlockSpec((NG, FF), lambda i: (0, 0)),
        ],
        out_shape=[
            jax.ShapeDtypeStruct((NG, FF), jnp.float32),
            jax.ShapeDtypeStruct((NG, FF), jnp.float32),
            jax.ShapeDtypeStruct((NG, FF), jnp.float32),
        ],
    )(parts, xw2, dinv, b2r, batch3d, wm, bmr)
    return out


def kernel(x, adj, batch, W1, b1, W2, b2, Wm, bm):
    src_flat = adj[0]
    dst_flat = adj[1]
    batch3d = batch.reshape(NB, 1, TCB)
    b1r = b1.reshape(1, FF)
    b2r = b2.reshape(1, FF)
    bmr = bm.reshape(1, FF)

    hist = _sc_deg(dst_flat)
    xw1 = _tc_matmul(x, W1)
    dinv, y1 = _tc_prep(hist, xw1)
    p1 = _sc_agg(y1, src_flat, dst_flat)
    xw2, y2 = _tc_combine1(p1, xw1, dinv, b1r, W2)
    p2 = _sc_agg(y2, src_flat, dst_flat)
    return _tc_pool(p2, xw2, dinv, b2r, batch3d, Wm, bmr)


# speedup vs baseline: 36.7533x; 1.0013x over previous
import dataclasses
import functools

import jax
import jax.numpy as jnp
from jax import lax
from jax.experimental import pallas as pl
from jax.experimental.pallas import tpu as pltpu
from jax.experimental.pallas import tpu_sc as plsc

NN = 10000
EE = 320000
FF = 128
NG = 64
NC, NS = 2, 16
WORKERS = NC * NS
CH = 80
DEPTH = 3
NCHUNK = EE // CH
CPW = NCHUNK // WORKERS
EPW = EE // WORKERS
RB = CH
TCB = 1000
NB = NN // TCB


def _vmesh():
    return plsc.VectorSubcoreMesh(core_axis_name="c", subcore_axis_name="s")


def _sc_params():
    cp = pltpu.CompilerParams()
    if "needs_layout_passes" in pltpu.CompilerParams.__dataclass_fields__:
        cp = dataclasses.replace(cp, needs_layout_passes=False)
    return cp


def _sc_deg(dst_flat):
    @functools.partial(
        pl.kernel,
        out_type=jax.ShapeDtypeStruct((WORKERS, NN), jnp.float32),
        mesh=_vmesh(),
        compiler_params=_sc_params(),
        scratch_types=[
            pltpu.VMEM((EPW,), jnp.int32),
            pltpu.VMEM((NN,), jnp.float32),
        ],
    )
    def k(dst_hbm, out_hbm, idx_v, hist_v):
        c = lax.axis_index("c")
        s = lax.axis_index("s")
        wid = s * NC + c

        @pl.loop(0, NN // 16)
        def _(i):
            hist_v[pl.ds(i * 16, 16)] = jnp.zeros((16,), jnp.float32)

        pltpu.sync_copy(dst_hbm.at[pl.ds(wid * EPW, EPW)], idx_v)
        ones = jnp.ones((16,), jnp.float32)

        @pl.loop(0, EPW // 16)
        def _(i):
            idx = idx_v[pl.ds(i * 16, 16)]
            plsc.addupdate_scatter(hist_v, [idx], ones)

        pltpu.sync_copy(hist_v, out_hbm.at[wid])

    return k(dst_flat)


def _sc_agg(y, src_flat, dst_flat):
    @functools.partial(
        pl.kernel,
        out_type=jax.ShapeDtypeStruct((NC, NN, FF), jnp.float32),
        mesh=_vmesh(),
        scratch_types=(
            [
                pltpu.VMEM((EPW,), jnp.int32),
                pltpu.VMEM((EPW,), jnp.int32),
            ]
            + [pltpu.VMEM((CH, FF), jnp.float32) for _ in range(DEPTH)]
            + [pltpu.SemaphoreType.DMA for _ in range(2 * DEPTH)]
            + [pltpu.VMEM_SHARED((NN, FF), jnp.float32)]
        ),
    )
    def k(y_hbm, src_hbm, dst_hbm, out_hbm, src_v, dst_v, *rest):
        bufs = rest[:DEPTH]
        sg = rest[DEPTH:2 * DEPTH]
        ss = rest[2 * DEPTH:3 * DEPTH]
        acc = rest[3 * DEPTH]
        c = lax.axis_index("c")
        s = lax.axis_index("s")
        wid = s * NC + c
        nz = jnp.where(s < 13, 8, 7)
        base = 640 * s - 80 * jnp.maximum(s - 13, 0)

        g0 = bufs[0]

        @pl.loop(0, RB)
        def _(r):
            @pl.loop(0, FF // 16)
            def _(q):
                g0[r, pl.ds(q * 16, 16)] = jnp.zeros((16,), jnp.float32)

        @pl.loop(0, 8)
        def _(t):
            @pl.when(t < nz)
            def _():
                pltpu.sync_copy(g0, acc.at[pl.ds(base + t * RB, RB)])

        plsc.subcore_barrier()

        pltpu.sync_copy(src_hbm.at[pl.ds(wid * EPW, EPW)], src_v)
        pltpu.sync_copy(dst_hbm.at[pl.ds(wid * EPW, EPW)], dst_v)

        def _src_at(j):
            return src_v.at[pl.ds(j * CH, CH)]

        def _dst_at(j):
            return dst_v.at[pl.ds(j * CH, CH)]

        def _step(j, b, refill, wait_prev):
            pltpu.make_async_copy(y_hbm.at[_src_at(j)], bufs[b], sg[b]).wait()
            pltpu.async_copy(bufs[b], acc.at[_dst_at(j)], ss[b], add=True)
            if refill:
                b2 = (b + 2) % DEPTH
                if wait_prev:
                    pltpu.make_async_copy(bufs[b2], acc.at[_dst_at(j)],
                                          ss[b2]).wait()
                pltpu.async_copy(y_hbm.at[_src_at(j + 2)], bufs[b2], sg[b2])

        pltpu.async_copy(y_hbm.at[_src_at(0)], bufs[0], sg[0])
        pltpu.async_copy(y_hbm.at[_src_at(1)], bufs[1], sg[1])
        for j in range(DEPTH - 2):
            _step(j, j, True, False)

        GROUPS = (CPW - DEPTH) // DEPTH
        @pl.loop(0, GROUPS)
        def _(t):
            for i in range(DEPTH):
                _step(t * DEPTH + (DEPTH - 2) + i, (DEPTH - 2 + i) % DEPTH,
                      True, True)

        for j in range(GROUPS * DEPTH + DEPTH - 2, CPW):
            _step(j, j % DEPTH, j + 2 < CPW, True)

        for j in range(CPW - DEPTH, CPW):
            pltpu.make_async_copy(bufs[j % DEPTH], acc.at[_dst_at(0)],
                                  ss[j % DEPTH]).wait()

        plsc.subcore_barrier()

        @pl.loop(0, 8)
        def _(t):
            @pl.when(t < nz)
            def _():
                pltpu.sync_copy(
                    acc.at[pl.ds(base + t * RB, RB)],
                    out_hbm.at[c, pl.ds(base + t * RB, RB)],
                )

    return k(y, src_flat, dst_flat)


def _tc_prep(hist, x, w1):
    PB = 1024

    def body(h_ref, x_ref, w_ref, dinv_ref, xw_ref, y_ref):
        deg = jnp.sum(h_ref[...], axis=0) + 1.0
        dinv = lax.rsqrt(deg)
        dinv_ref[...] = dinv[:, None]
        xw = jnp.dot(x_ref[...], w_ref[...], preferred_element_type=jnp.float32)
        xw_ref[...] = xw
        y_ref[...] = dinv[:, None] * xw

    return pl.pallas_call(
        body,
        grid=(NB,),
        in_specs=[
            pl.BlockSpec((WORKERS, PB), lambda i: (0, i)),
            pl.BlockSpec((PB, FF), lambda i: (i, 0)),
            pl.BlockSpec((FF, FF), lambda i: (0, 0)),
        ],
        out_specs=[
            pl.BlockSpec((PB, 1), lambda i: (i, 0)),
            pl.BlockSpec((PB, FF), lambda i: (i, 0)),
            pl.BlockSpec((PB, FF), lambda i: (i, 0)),
        ],
        out_shape=[
            jax.ShapeDtypeStruct((NN, 1), jnp.float32),
            jax.ShapeDtypeStruct((NN, FF), jnp.float32),
            jax.ShapeDtypeStruct((NN, FF), jnp.float32),
        ],
    )(hist, x, w1)


def _tc_combine1(parts, xw1, dinv, b1r, w2):
    def body(p_ref, xw_ref, d_ref, b_ref, w_ref, xw2_ref, y2_ref):
        dinv = d_ref[...]
        h = dinv * (p_ref[0] + p_ref[1]) + dinv * dinv * xw_ref[...] + b_ref[...]
        h = jnp.maximum(h, 0.0)
        xw2 = jnp.dot(h, w_ref[...], preferred_element_type=jnp.float32)
        xw2_ref[...] = xw2
        y2_ref[...] = dinv * xw2

    return pl.pallas_call(
        body,
        grid=(NB,),
        in_specs=[
            pl.BlockSpec((NC, TCB, FF), lambda i: (0, i, 0)),
            pl.BlockSpec((TCB, FF), lambda i: (i, 0)),
            pl.BlockSpec((TCB, 1), lambda i: (i, 0)),
            pl.BlockSpec((1, FF), lambda i: (0, 0)),
            pl.BlockSpec((FF, FF), lambda i: (0, 0)),
        ],
        out_specs=[
            pl.BlockSpec((TCB, FF), lambda i: (i, 0)),
            pl.BlockSpec((TCB, FF), lambda i: (i, 0)),
        ],
        out_shape=[
            jax.ShapeDtypeStruct((NN, FF), jnp.float32),
            jax.ShapeDtypeStruct((NN, FF), jnp.float32),
        ],
    )(parts, xw1, dinv, b1r, w2)


def _tc_pool(parts, xw2, dinv, b2r, batch3d, wm, bmr):
    def body(p_ref, xw_ref, d_ref, b_ref, bt_ref, wm_ref, bm_ref,
             o_ref, sums_ref, cnts_ref):
        i = pl.program_id(0)
        dinv = d_ref[...]
        h2 = dinv * (p_ref[0] + p_ref[1]) + dinv * dinv * xw_ref[...] + b_ref[...]
        bt = bt_ref[0, 0, :]
        gids = lax.broadcasted_iota(jnp.int32, (1, NG), 1)
        onehot = (bt[:, None] == gids).astype(jnp.float32)
        contrib = lax.dot_general(onehot, h2, (((0,), (0,)), ((), ())),
                                  preferred_element_type=jnp.float32)
        cnt = lax.dot_general(onehot, jnp.ones((TCB, FF), jnp.float32),
                              (((0,), (0,)), ((), ())),
                              preferred_element_type=jnp.float32)

        @pl.when(i == 0)
        def _():
            sums_ref[...] = jnp.zeros_like(sums_ref)
            cnts_ref[...] = jnp.zeros_like(cnts_ref)

        sums_ref[...] += contrib
        cnts_ref[...] += cnt

        @pl.when(i == NB - 1)
        def _():
            pooled = sums_ref[...] / jnp.maximum(cnts_ref[...], 1.0)
            o_ref[...] = jnp.dot(pooled, wm_ref[...],
                                 preferred_element_type=jnp.float32) + bm_ref[...]

    out, _, _ = pl.pallas_call(
        body,
        grid=(NB,),
        in_specs=[
            pl.BlockSpec((NC, TCB, FF), lambda i: (0, i, 0)),
            pl.BlockSpec((TCB, FF), lambda i: (i, 0)),
            pl.BlockSpec((TCB, 1), lambda i: (i, 0)),
            pl.BlockSpec((1, FF), lambda i: (0, 0)),
            pl.BlockSpec((1, 1, TCB), lambda i: (i, 0, 0)),
            pl.BlockSpec((FF, FF), lambda i: (0, 0)),
            pl.BlockSpec((1, FF), lambda i: (0, 0)),
        ],
        out_specs=[
            pl.BlockSpec((NG, FF), lambda i: (0, 0)),
            pl.BlockSpec((NG, FF), lambda i: (0, 0)),
            pl.BlockSpec((NG, FF), lambda i: (0, 0)),
        ],
        out_shape=[
            jax.ShapeDtypeStruct((NG, FF), jnp.float32),
            jax.ShapeDtypeStruct((NG, FF), jnp.float32),
            jax.ShapeDtypeStruct((NG, FF), jnp.float32),
        ],
    )(parts, xw2, dinv, b2r, batch3d, wm, bmr)
    return out


def kernel(x, adj, batch, W1, b1, W2, b2, Wm, bm):
    src_flat = adj[0]
    dst_flat = adj[1]
    batch3d = batch.reshape(NB, 1, TCB)
    b1r = b1.reshape(1, FF)
    b2r = b2.reshape(1, FF)
    bmr = bm.reshape(1, FF)

    hist = _sc_deg(dst_flat)
    dinv, xw1, y1 = _tc_prep(hist, x, W1)
    p1 = _sc_agg(y1, src_flat, dst_flat)
    xw2, y2 = _tc_combine1(p1, xw1, dinv, b1r, W2)
    p2 = _sc_agg(y2, src_flat, dst_flat)
    return _tc_pool(p2, xw2, dinv, b2r, batch3d, Wm, bmr)


# speedup vs baseline: 37.0464x; 1.0080x over previous
import dataclasses
import functools

import jax
import jax.numpy as jnp
from jax import lax
from jax.experimental import pallas as pl
from jax.experimental.pallas import tpu as pltpu
from jax.experimental.pallas import tpu_sc as plsc

NN = 10000
EE = 320000
FF = 128
NG = 64
NC, NS = 2, 16
WORKERS = NC * NS
CH = 80
DEPTH = 3
NCHUNK = EE // CH
CPW = NCHUNK // WORKERS
EPW = EE // WORKERS
RB = CH
TCB = 1000
NB = NN // TCB


def _vmesh():
    return plsc.VectorSubcoreMesh(core_axis_name="c", subcore_axis_name="s")


def _sc_params():
    cp = pltpu.CompilerParams()
    if "needs_layout_passes" in pltpu.CompilerParams.__dataclass_fields__:
        cp = dataclasses.replace(cp, needs_layout_passes=False)
    return cp


def _sc_deg(dst_flat):
    @functools.partial(
        pl.kernel,
        out_type=jax.ShapeDtypeStruct((WORKERS, NN), jnp.float32),
        mesh=_vmesh(),
        compiler_params=_sc_params(),
        scratch_types=[
            pltpu.VMEM((EPW,), jnp.int32),
            pltpu.VMEM((NN,), jnp.float32),
        ],
    )
    def k(dst_hbm, out_hbm, idx_v, hist_v):
        c = lax.axis_index("c")
        s = lax.axis_index("s")
        wid = s * NC + c

        @pl.loop(0, NN // 16)
        def _(i):
            hist_v[pl.ds(i * 16, 16)] = jnp.zeros((16,), jnp.float32)

        pltpu.sync_copy(dst_hbm.at[pl.ds(wid * EPW, EPW)], idx_v)
        ones = jnp.ones((16,), jnp.float32)

        @pl.loop(0, EPW // 16)
        def _(i):
            idx = idx_v[pl.ds(i * 16, 16)]
            plsc.addupdate_scatter(hist_v, [idx], ones)

        pltpu.sync_copy(hist_v, out_hbm.at[wid])

    return k(dst_flat)


def _sc_agg(y, src_flat, dst_flat):
    @functools.partial(
        pl.kernel,
        out_type=jax.ShapeDtypeStruct((NC, NN, FF), jnp.float32),
        mesh=_vmesh(),
        scratch_types=(
            [
                pltpu.VMEM((EPW,), jnp.int32),
                pltpu.VMEM((EPW,), jnp.int32),
            ]
            + [pltpu.VMEM((CH, FF), jnp.float32) for _ in range(DEPTH)]
            + [pltpu.SemaphoreType.DMA for _ in range(2 * DEPTH)]
            + [pltpu.VMEM_SHARED((NN, FF), jnp.float32)]
        ),
    )
    def k(y_hbm, src_hbm, dst_hbm, out_hbm, src_v, dst_v, *rest):
        bufs = rest[:DEPTH]
        sg = rest[DEPTH:2 * DEPTH]
        ss = rest[2 * DEPTH:3 * DEPTH]
        acc = rest[3 * DEPTH]
        c = lax.axis_index("c")
        s = lax.axis_index("s")
        wid = s * NC + c
        nz = jnp.where(s < 13, 8, 7)
        base = 640 * s - 80 * jnp.maximum(s - 13, 0)

        pltpu.sync_copy(src_hbm.at[pl.ds(wid * EPW, EPW)], src_v)
        pltpu.sync_copy(dst_hbm.at[pl.ds(wid * EPW, EPW)], dst_v)

        def _src_at(j):
            return src_v.at[pl.ds(j * CH, CH)]

        def _dst_at(j):
            return dst_v.at[pl.ds(j * CH, CH)]

        pltpu.async_copy(y_hbm.at[_src_at(0)], bufs[0], sg[0])
        pltpu.async_copy(y_hbm.at[_src_at(1)], bufs[1], sg[1])

        zb = bufs[DEPTH - 1]

        @pl.loop(0, RB)
        def _(r):
            @pl.loop(0, FF // 16)
            def _(q):
                zb[r, pl.ds(q * 16, 16)] = jnp.zeros((16,), jnp.float32)

        @pl.loop(0, 8)
        def _(t):
            @pl.when(t < nz)
            def _():
                pltpu.sync_copy(zb, acc.at[pl.ds(base + t * RB, RB)])

        plsc.subcore_barrier()

        def _step(j, b, refill, wait_prev):
            pltpu.make_async_copy(y_hbm.at[_src_at(j)], bufs[b], sg[b]).wait()
            pltpu.async_copy(bufs[b], acc.at[_dst_at(j)], ss[b], add=True)
            if refill:
                b2 = (b + 2) % DEPTH
                if wait_prev:
                    pltpu.make_async_copy(bufs[b2], acc.at[_dst_at(j)],
                                          ss[b2]).wait()
                pltpu.async_copy(y_hbm.at[_src_at(j + 2)], bufs[b2], sg[b2])

        for j in range(DEPTH - 2):
            _step(j, j, True, False)

        GROUPS = (CPW - DEPTH) // DEPTH
        @pl.loop(0, GROUPS)
        def _(t):
            for i in range(DEPTH):
                _step(t * DEPTH + (DEPTH - 2) + i, (DEPTH - 2 + i) % DEPTH,
                      True, True)

        for j in range(GROUPS * DEPTH + DEPTH - 2, CPW):
            _step(j, j % DEPTH, j + 2 < CPW, True)

        for j in range(CPW - DEPTH, CPW):
            pltpu.make_async_copy(bufs[j % DEPTH], acc.at[_dst_at(0)],
                                  ss[j % DEPTH]).wait()

        plsc.subcore_barrier()

        @pl.loop(0, 8)
        def _(t):
            @pl.when(t < nz)
            def _():
                pltpu.sync_copy(
                    acc.at[pl.ds(base + t * RB, RB)],
                    out_hbm.at[c, pl.ds(base + t * RB, RB)],
                )

    return k(y, src_flat, dst_flat)


def _tc_prep(hist, x, w1):
    PB = 1024

    def body(h_ref, x_ref, w_ref, dinv_ref, xw_ref, y_ref):
        deg = jnp.sum(h_ref[...], axis=0) + 1.0
        dinv = lax.rsqrt(deg)
        dinv_ref[...] = dinv[:, None]
        xw = jnp.dot(x_ref[...], w_ref[...], preferred_element_type=jnp.float32)
        xw_ref[...] = xw
        y_ref[...] = dinv[:, None] * xw

    return pl.pallas_call(
        body,
        grid=(NB,),
        in_specs=[
            pl.BlockSpec((WORKERS, PB), lambda i: (0, i)),
            pl.BlockSpec((PB, FF), lambda i: (i, 0)),
            pl.BlockSpec((FF, FF), lambda i: (0, 0)),
        ],
        out_specs=[
            pl.BlockSpec((PB, 1), lambda i: (i, 0)),
            pl.BlockSpec((PB, FF), lambda i: (i, 0)),
            pl.BlockSpec((PB, FF), lambda i: (i, 0)),
        ],
        out_shape=[
            jax.ShapeDtypeStruct((NN, 1), jnp.float32),
            jax.ShapeDtypeStruct((NN, FF), jnp.float32),
            jax.ShapeDtypeStruct((NN, FF), jnp.float32),
        ],
    )(hist, x, w1)


def _tc_combine1(parts, xw1, dinv, b1r, w2):
    def body(p_ref, xw_ref, d_ref, b_ref, w_ref, xw2_ref, y2_ref):
        dinv = d_ref[...]
        h = dinv * (p_ref[0] + p_ref[1]) + dinv * dinv * xw_ref[...] + b_ref[...]
        h = jnp.maximum(h, 0.0)
        xw2 = jnp.dot(h, w_ref[...], preferred_element_type=jnp.float32)
        xw2_ref[...] = xw2
        y2_ref[...] = dinv * xw2

    return pl.pallas_call(
        body,
        grid=(NB,),
        in_specs=[
            pl.BlockSpec((NC, TCB, FF), lambda i: (0, i, 0)),
            pl.BlockSpec((TCB, FF), lambda i: (i, 0)),
            pl.BlockSpec((TCB, 1), lambda i: (i, 0)),
            pl.BlockSpec((1, FF), lambda i: (0, 0)),
            pl.BlockSpec((FF, FF), lambda i: (0, 0)),
        ],
        out_specs=[
            pl.BlockSpec((TCB, FF), lambda i: (i, 0)),
            pl.BlockSpec((TCB, FF), lambda i: (i, 0)),
        ],
        out_shape=[
            jax.ShapeDtypeStruct((NN, FF), jnp.float32),
            jax.ShapeDtypeStruct((NN, FF), jnp.float32),
        ],
    )(parts, xw1, dinv, b1r, w2)


def _tc_pool(parts, xw2, dinv, b2r, batch3d, wm, bmr):
    def body(p_ref, xw_ref, d_ref, b_ref, bt_ref, wm_ref, bm_ref,
             o_ref, sums_ref, cnts_ref):
        i = pl.program_id(0)
        dinv = d_ref[...]
        h2 = dinv * (p_ref[0] + p_ref[1]) + dinv * dinv * xw_ref[...] + b_ref[...]
        bt = bt_ref[0, 0, :]
        gids = lax.broadcasted_iota(jnp.int32, (1, NG), 1)
        onehot = (bt[:, None] == gids).astype(jnp.float32)
        contrib = lax.dot_general(onehot, h2, (((0,), (0,)), ((), ())),
                                  preferred_element_type=jnp.float32)
        cnt = lax.dot_general(onehot, jnp.ones((TCB, FF), jnp.float32),
                              (((0,), (0,)), ((), ())),
                              preferred_element_type=jnp.float32)

        @pl.when(i == 0)
        def _():
            sums_ref[...] = jnp.zeros_like(sums_ref)
            cnts_ref[...] = jnp.zeros_like(cnts_ref)

        sums_ref[...] += contrib
        cnts_ref[...] += cnt

        @pl.when(i == NB - 1)
        def _():
            pooled = sums_ref[...] / jnp.maximum(cnts_ref[...], 1.0)
            o_ref[...] = jnp.dot(pooled, wm_ref[...],
                                 preferred_element_type=jnp.float32) + bm_ref[...]

    out, _, _ = pl.pallas_call(
        body,
        grid=(NB,),
        in_specs=[
            pl.BlockSpec((NC, TCB, FF), lambda i: (0, i, 0)),
            pl.BlockSpec((TCB, FF), lambda i: (i, 0)),
            pl.BlockSpec((TCB, 1), lambda i: (i, 0)),
            pl.BlockSpec((1, FF), lambda i: (0, 0)),
            pl.BlockSpec((1, 1, TCB), lambda i: (i, 0, 0)),
            pl.BlockSpec((FF, FF), lambda i: (0, 0)),
            pl.BlockSpec((1, FF), lambda i: (0, 0)),
        ],
        out_specs=[
            pl.BlockSpec((NG, FF), lambda i: (0, 0)),
            pl.BlockSpec((NG, FF), lambda i: (0, 0)),
            pl.BlockSpec((NG, FF), lambda i: (0, 0)),
        ],
        out_shape=[
            jax.ShapeDtypeStruct((NG, FF), jnp.float32),
            jax.ShapeDtypeStruct((NG, FF), jnp.float32),
            jax.ShapeDtypeStruct((NG, FF), jnp.float32),
        ],
    )(parts, xw2, dinv, b2r, batch3d, wm, bmr)
    return out


def kernel(x, adj, batch, W1, b1, W2, b2, Wm, bm):
    src_flat = adj[0]
    dst_flat = adj[1]
    batch3d = batch.reshape(NB, 1, TCB)
    b1r = b1.reshape(1, FF)
    b2r = b2.reshape(1, FF)
    bmr = bm.reshape(1, FF)

    hist = _sc_deg(dst_flat)
    dinv, xw1, y1 = _tc_prep(hist, x, W1)
    p1 = _sc_agg(y1, src_flat, dst_flat)
    xw2, y2 = _tc_combine1(p1, xw1, dinv, b1r, W2)
    p2 = _sc_agg(y2, src_flat, dst_flat)
    return _tc_pool(p2, xw2, dinv, b2r, batch3d, Wm, bmr)
